# Initial kernel scaffold; baseline (speedup 1.0000x reference)
import functools

import jax
import jax.numpy as jnp
from jax import lax
from jax.experimental import pallas as pl
from jax.experimental.pallas import tpu as pltpu
from jax.experimental.pallas import tpu_sc as plsc

B, S, D = 4, 8192, 768
DINUC_DIM = D // 4
EPS = 1e-12
TOK = B * S
NC, NS = 2, 16
NW = NC * NS
BPW = TOK // NW
CH = 128
NCH = BPW // CH


def _prep_body(ids_ref, tt_ref, dt_ref, w_ref, b_ref, g_ref, be_ref,
               lut_ref, c_ref):
    w_top = w_ref[:D, :]
    w_bot = w_ref[D:, :]
    tf = jnp.dot(tt_ref[...], w_top, preferred_element_type=jnp.float32)
    df = jnp.dot(dt_ref[...], w_bot, preferred_element_type=jnp.float32)
    df32 = jnp.concatenate([df, jnp.zeros((16, D), jnp.float32)], axis=0)
    x = tf[:, None, :] + df32[None, :, :] + b_ref[...][None, :, :]
    mean = jnp.mean(x, axis=-1, keepdims=True)
    var = jnp.mean((x - mean) ** 2, axis=-1, keepdims=True)
    lut_ref[...] = ((x - mean) * lax.rsqrt(var + EPS)
                    * g_ref[...][None, :, :] + be_ref[...][None, :, :])

    first = ids_ref[...]
    second = jnp.concatenate(
        [first[:, 1:], jnp.zeros((B, 1), jnp.int32)], axis=1)
    valid = ((first >= 4) & (first <= 7) & (second >= 4) & (second <= 7))
    did = jnp.where(valid, (first - 4) * 4 + (second - 4), 0)
    col = lax.broadcasted_iota(jnp.int32, (B, S), 1)
    did = jnp.where(col == S - 1, 16, did)
    c_ref[...] = first * 32 + did


def _prep(input_ids, token_table, dinuc_table, proj_w, proj_b, ln_gamma,
          ln_beta):
    return pl.pallas_call(
        _prep_body,
        out_shape=(
            jax.ShapeDtypeStruct((8, 32, D), jnp.float32),
            jax.ShapeDtypeStruct((B, S), jnp.int32),
        ),
    )(input_ids, token_table, dinuc_table, proj_w,
      proj_b.reshape(1, D), ln_gamma.reshape(1, D), ln_beta.reshape(1, D))


def _sc_gather_body(lut_hbm, idx_hbm, out_hbm, idx_v, buf0, buf1, sem0, sem1):
    wid = lax.axis_index("s") * NC + lax.axis_index("c")
    base = wid * BPW
    pltpu.sync_copy(idx_hbm.at[pl.ds(base, BPW)], idx_v)
    bufs = (buf0, buf1)
    sems = (sem0, sem1)
    cps = [None, None]
    cps[0] = pltpu.async_copy(lut_hbm.at[idx_v.at[pl.ds(0, CH)]], buf0, sem0)
    for k in range(NCH):
        if k + 1 < NCH:
            cps[(k + 1) % 2] = pltpu.async_copy(
                lut_hbm.at[idx_v.at[pl.ds((k + 1) * CH, CH)]],
                bufs[(k + 1) % 2], sems[(k + 1) % 2])
        cps[k % 2].wait()
        pltpu.sync_copy(bufs[k % 2], out_hbm.at[pl.ds(base + k * CH, CH)])


_sc_gather = functools.partial(
    pl.kernel,
    out_type=jax.ShapeDtypeStruct((TOK, D), jnp.float32),
    mesh=plsc.VectorSubcoreMesh(core_axis_name="c", subcore_axis_name="s"),
    scratch_types=[
        pltpu.VMEM((BPW,), jnp.int32),
        pltpu.VMEM((CH, D), jnp.float32),
        pltpu.VMEM((CH, D), jnp.float32),
        pltpu.SemaphoreType.DMA,
        pltpu.SemaphoreType.DMA,
    ],
)(_sc_gather_body)


@jax.jit
def kernel(input_ids, token_table, dinuc_table, proj_w, proj_b, ln_gamma,
           ln_beta):
    lut, c = _prep(input_ids, token_table, dinuc_table, proj_w, proj_b,
                   ln_gamma, ln_beta)
    out = _sc_gather(lut.reshape(8 * 32, D), c.reshape(TOK))
    return out.reshape(B, S, D)


# speedup vs baseline: 1.2721x; 1.2721x over previous
import functools

import jax
import jax.numpy as jnp
from jax import lax
from jax.experimental import pallas as pl
from jax.experimental.pallas import tpu as pltpu
from jax.experimental.pallas import tpu_sc as plsc

B, S, D = 4, 8192, 768
DINUC_DIM = D // 4
EPS = 1e-12
TOK = B * S
NC, NS = 2, 16
NW = NC * NS
BPW = TOK // NW
CH = 64
NCH = BPW // CH


def _prep_body(ids_ref, tt_ref, dt_ref, w_ref, b_ref, g_ref, be_ref,
               lut_ref, c_ref):
    w_top = w_ref[:D, :]
    w_bot = w_ref[D:, :]
    tf = jnp.dot(tt_ref[...], w_top, preferred_element_type=jnp.float32)
    df = jnp.dot(dt_ref[...], w_bot, preferred_element_type=jnp.float32)
    df32 = jnp.concatenate([df, jnp.zeros((16, D), jnp.float32)], axis=0)
    x = tf[:, None, :] + df32[None, :, :] + b_ref[...][None, :, :]
    mean = jnp.mean(x, axis=-1, keepdims=True)
    var = jnp.mean((x - mean) ** 2, axis=-1, keepdims=True)
    lut_ref[...] = ((x - mean) * lax.rsqrt(var + EPS)
                    * g_ref[...][None, :, :] + be_ref[...][None, :, :])

    first = ids_ref[...]
    second = jnp.concatenate(
        [first[:, 1:], jnp.zeros((B, 1), jnp.int32)], axis=1)
    valid = ((first >= 4) & (first <= 7) & (second >= 4) & (second <= 7))
    did = jnp.where(valid, (first - 4) * 4 + (second - 4), 0)
    col = lax.broadcasted_iota(jnp.int32, (B, S), 1)
    did = jnp.where(col == S - 1, 16, did)
    c_ref[...] = first * 32 + did


def _prep(input_ids, token_table, dinuc_table, proj_w, proj_b, ln_gamma,
          ln_beta):
    return pl.pallas_call(
        _prep_body,
        out_shape=(
            jax.ShapeDtypeStruct((8, 32, D), jnp.float32),
            jax.ShapeDtypeStruct((B, S), jnp.int32),
        ),
    )(input_ids, token_table, dinuc_table, proj_w,
      proj_b.reshape(1, D), ln_gamma.reshape(1, D), ln_beta.reshape(1, D))


def _sc_gather_body(lut_hbm, idx_hbm, out_hbm, idx_v, buf0, buf1, sem0, sem1):
    wid = lax.axis_index("s") * NC + lax.axis_index("c")
    base = wid * BPW
    pltpu.sync_copy(idx_hbm.at[pl.ds(base, BPW)], idx_v)
    bufs = (buf0, buf1)
    sems = (sem0, sem1)
    cps = [None, None]
    cps[0] = pltpu.async_copy(lut_hbm.at[idx_v.at[pl.ds(0, CH)]], buf0, sem0)
    for k in range(NCH):
        if k + 1 < NCH:
            cps[(k + 1) % 2] = pltpu.async_copy(
                lut_hbm.at[idx_v.at[pl.ds((k + 1) * CH, CH)]],
                bufs[(k + 1) % 2], sems[(k + 1) % 2])
        cps[k % 2].wait()
        pltpu.sync_copy(bufs[k % 2], out_hbm.at[pl.ds(base + k * CH, CH)])


@functools.cache
def _sc_gather():
    return pl.kernel(
        _sc_gather_body,
        out_type=jax.ShapeDtypeStruct((TOK, D), jnp.float32),
        mesh=plsc.VectorSubcoreMesh(core_axis_name="c", subcore_axis_name="s",
                                    num_cores=NC, num_subcores=NS),
        scratch_types=[
            pltpu.VMEM((BPW,), jnp.int32),
            pltpu.VMEM((CH, D), jnp.float32),
            pltpu.VMEM((CH, D), jnp.float32),
            pltpu.SemaphoreType.DMA,
            pltpu.SemaphoreType.DMA,
        ],
    )


@jax.jit
def kernel(input_ids, token_table, dinuc_table, proj_w, proj_b, ln_gamma,
           ln_beta):
    lut, c = _prep(input_ids, token_table, dinuc_table, proj_w, proj_b,
                   ln_gamma, ln_beta)
    out = _sc_gather()(lut.reshape(8 * 32, D), c.reshape(TOK))
    return out.reshape(B, S, D)
